# Initial kernel scaffold; baseline (speedup 1.0000x reference)
import functools

import jax
import jax.numpy as jnp
from jax import lax
from jax.experimental import pallas as pl
from jax.experimental.pallas import tpu as pltpu
from jax.experimental.pallas import tpu_sc as plsc

H = 192
W = 256
BQ = 2
NPTS = 262144
SIZE = BQ * H * W
PN = BQ * NPTS
POSMASK = (1 << 19) - 1
I32MAX = 2**31 - 1

NC, NS = 2, 16
NW = NC * NS
PTS_PER_W = PN // NW
CHUNK = 4096
SLICE_M = SIZE // NS
SLICE_O = SIZE // NW
NROW = SLICE_O // 128

NRESH = 2048
RB = 64


def _batcher_pairs(n):
    p = 1
    while p < n:
        p *= 2
    pairs = []

    def merge(lo, hi, r):
        step = r * 2
        if step < hi - lo:
            merge(lo, hi, step)
            merge(lo + r, hi, step)
            for i in range(lo + r, hi - r, step):
                pairs.append((i, i + r))
        else:
            pairs.append((lo, lo + r))

    def sort(lo, hi):
        if (hi - lo) >= 1:
            mid = lo + ((hi - lo) // 2)
            sort(lo, mid)
            sort(mid + 1, hi)
            merge(lo, hi, 1)

    sort(0, p - 1)
    return [(i, j) for (i, j) in pairs if i < n and j < n]


def _prune(pairs, out_idx):
    needed = {out_idx}
    ops = []
    for (i, j) in reversed(pairs):
        ni, nj = i in needed, j in needed
        if not ni and not nj:
            continue
        ops.append(('ce' if (ni and nj) else ('min' if ni else 'max'), i, j))
        needed.add(i)
        needed.add(j)
    ops.reverse()
    return ops


_MED_OPS = _prune(_batcher_pairs(25), 12)


def _proj_body(mats_ref, cloud_ref, sk_ref, fk_ref):
    b = pl.program_id(0)
    j = pl.program_id(1)
    x = cloud_ref[0, 0]
    y = cloud_ref[0, 1]
    z = cloud_ref[0, 2]

    def m(i, k):
        return mats_ref[0, i, k]

    cam0 = x * m(0, 0) + y * m(0, 1) + z * m(0, 2) + m(0, 3)
    cam1 = x * m(1, 0) + y * m(1, 1) + z * m(1, 2) + m(1, 3)
    cam2 = x * m(2, 0) + y * m(2, 1) + z * m(2, 2) + m(2, 3)
    p0 = cam0 * m(4, 0) + cam1 * m(4, 1) + cam2 * m(4, 2)
    p1 = cam0 * m(5, 0) + cam1 * m(5, 1) + cam2 * m(5, 2)
    p2 = cam0 * m(6, 0) + cam1 * m(6, 1) + cam2 * m(6, 2)
    u = jnp.clip((p0 / p2).astype(jnp.int32), 0, W - 1)
    v = jnp.clip((p1 / p2).astype(jnp.int32), 0, H - 1)
    pix = b * (H * W) + v * W + u
    di = cam2.astype(jnp.int32)
    row = j * RB + lax.broadcasted_iota(jnp.int32, (RB, 128), 0)
    col = lax.broadcasted_iota(jnp.int32, (RB, 128), 1)
    n = row * 128 + col
    posf = (PN - 1) - (b * NPTS + n)
    fk_ref[0] = jnp.clip(di, 0, 4094) * 524288 + posf
    lane_posf = 15 - (col % 16)
    sk_ref[0] = pix * 512 + jnp.clip(di, 0, 31) * 16 + lane_posf


def _proj_call(mats, cloud4, interpret=False):
    return pl.pallas_call(
        _proj_body,
        grid=(BQ, NRESH // RB),
        in_specs=[
            pl.BlockSpec((1, 8, 4), lambda b, j: (b, 0, 0),
                         memory_space=pltpu.SMEM),
            pl.BlockSpec((1, 4, RB, 128), lambda b, j: (b, 0, j, 0)),
        ],
        out_specs=[
            pl.BlockSpec((1, RB, 128), lambda b, j: (b, j, 0)),
            pl.BlockSpec((1, RB, 128), lambda b, j: (b, j, 0)),
        ],
        out_shape=[
            jax.ShapeDtypeStruct((BQ, NRESH, 128), jnp.int32),
            jax.ShapeDtypeStruct((BQ, NRESH, 128), jnp.int32),
        ],
        interpret=interpret,
    )(mats, cloud4)


_MESH = plsc.VectorSubcoreMesh(core_axis_name="c", subcore_axis_name="s")


@functools.partial(
    pl.kernel,
    out_type=jax.ShapeDtypeStruct((NC, SIZE), jnp.int32),
    mesh=_MESH,
    scratch_types=[
        pltpu.VMEM((SIZE,), jnp.int32),
        pltpu.VMEM((CHUNK,), jnp.int32),
        pltpu.VMEM((CHUNK,), jnp.int32),
        pltpu.VMEM((16,), jnp.int32),
        pltpu.VMEM((SLICE_M,), jnp.int32),
        pltpu.VMEM((SLICE_M,), jnp.int32),
        pltpu.VMEM_SHARED((NS, SIZE), jnp.int32),
    ],
)
def _scatter_k(sk_hbm, fk_hbm, out_hbm, zbuf, ssk, sfk, svec, acc, tmp, part):
    cid = lax.axis_index("c")
    sid = lax.axis_index("s")
    wid = cid * NS + sid
    iota = lax.broadcasted_iota(jnp.int32, (16,), 0)
    idxm1 = jnp.maximum(iota - 1, 0)
    maxv = jnp.full((16,), I32MAX, jnp.int32)

    def init_body(i, _):
        zbuf[pl.ds(i * 16, 16)] = maxv
        return 0

    lax.fori_loop(0, SIZE // 16, init_body, 0)

    base = wid * PTS_PER_W
    for ch in range(PTS_PER_W // CHUNK):
        off = base + ch * CHUNK
        pltpu.sync_copy(sk_hbm.at[pl.ds(off, CHUNK)], ssk)
        pltpu.sync_copy(fk_hbm.at[pl.ds(off, CHUNK)], sfk)

        def pt_body(i, _):
            sk = ssk[pl.ds(i * 16, 16)]
            fk = sfk[pl.ds(i * 16, 16)]
            k_s, v_s = plsc.sort_key_val(sk, fk)
            svec[...] = k_s
            prev = plsc.load_gather(svec, [idxm1])
            spix = lax.shift_right_logical(k_s, 9)
            ppix = lax.shift_right_logical(prev, 9)
            first = (spix != ppix) | (iota == 0)
            cur = plsc.load_gather(zbuf, [spix])
            newv = jnp.minimum(cur, v_s)
            plsc.store_scatter(zbuf, [spix], newv, mask=first)
            return 0

        lax.fori_loop(0, CHUNK // 16, pt_body, 0)

    pltpu.sync_copy(zbuf, part.at[sid])
    plsc.subcore_barrier()

    ms = sid * SLICE_M
    pltpu.sync_copy(part.at[0, pl.ds(ms, SLICE_M)], acc)
    for jj in range(1, NS):
        pltpu.sync_copy(part.at[jj, pl.ds(ms, SLICE_M)], tmp)

        def merge_body(i, _):
            acc[pl.ds(i * 16, 16)] = jnp.minimum(
                acc[pl.ds(i * 16, 16)], tmp[pl.ds(i * 16, 16)])
            return 0

        lax.fori_loop(0, SLICE_M // 16, merge_body, 0)

    pltpu.sync_copy(acc, out_hbm.at[cid, pl.ds(ms, SLICE_M)])


@functools.partial(
    pl.kernel,
    out_type=jax.ShapeDtypeStruct((BQ * 3 * H * W,), jnp.float32),
    mesh=_MESH,
    scratch_types=[
        pltpu.VMEM((SLICE_O,), jnp.int32),
        pltpu.VMEM((SLICE_O,), jnp.int32),
        pltpu.VMEM((3, NROW, 128), jnp.int32),
        pltpu.VMEM((3, NROW, 128), jnp.float32),
        pltpu.VMEM((3, SLICE_O), jnp.float32),
        pltpu.SemaphoreType.DMA,
    ],
)
def _winner_k(p_hbm, rgb_hbm, img_hbm, k0, k1, idxr, gat, ost, sem):
    cid = lax.axis_index("c")
    sid = lax.axis_index("s")
    wid = cid * NS + sid
    pb = wid * SLICE_O
    b = wid // NS
    pltpu.sync_copy(p_hbm.at[0, pl.ds(pb, SLICE_O)], k0)
    pltpu.sync_copy(p_hbm.at[1, pl.ds(pb, SLICE_O)], k1)
    iota = lax.broadcasted_iota(jnp.int32, (16,), 0)

    def idx_body(r, _):
        for jj in range(8):
            i = r * 8 + jj
            a = jnp.minimum(k0[pl.ds(i * 16, 16)], k1[pl.ds(i * 16, 16)])
            k0[pl.ds(i * 16, 16)] = a
            valid = a != I32MAX
            posf = a & POSMASK
            n = (PN - 1) - posf - b * NPTS
            spread = (pb + i * 16 + iota) & 1023
            base_i = b * (3 * NPTS) + n
            for c in range(3):
                idxr[c, r, pl.ds(jj * 16, 16)] = jnp.where(
                    valid, base_i + c * NPTS, spread)
        return 0

    lax.fori_loop(0, NROW, idx_body, 0)

    for c in range(3):
        copies = [
            pltpu.async_copy(rgb_hbm.at[idxr.at[c, r]], gat.at[c, r], sem)
            for r in range(NROW)
        ]
        for cp in copies:
            cp.wait()

    def out_body(r, _):
        for jj in range(8):
            i = r * 8 + jj
            a = k0[pl.ds(i * 16, 16)]
            valid = a != I32MAX
            for c in range(3):
                g = gat[c, r, pl.ds(jj * 16, 16)]
                ost[c, pl.ds(i * 16, 16)] = jnp.where(valid, g, -0.001)
        return 0

    lax.fori_loop(0, NROW, out_body, 0)

    pwb = pb - b * (H * W)
    for c in range(3):
        dst = b * (3 * H * W) + c * (H * W) + pwb
        pltpu.sync_copy(ost.at[c], img_hbm.at[pl.ds(dst, SLICE_O)])


def _median_body(img_ref, out_ref):
    img = img_ref[0]
    mask3 = jnp.broadcast_to(img[0:1] > 0, (3, H, W))

    def pad_reflect(x):
        x = jnp.concatenate(
            [x[:, 2:3], x[:, 1:2], x, x[:, H - 2:H - 1], x[:, H - 3:H - 2]],
            axis=1)
        x = jnp.concatenate(
            [x[:, :, 2:3], x[:, :, 1:2], x,
             x[:, :, W - 2:W - 1], x[:, :, W - 3:W - 2]],
            axis=2)
        return x

    def median25(x):
        p = pad_reflect(x)
        v = [p[:, di:di + H, dj:dj + W] for di in range(5) for dj in range(5)]
        for kind, i, j in _MED_OPS:
            if kind == 'ce':
                v[i], v[j] = jnp.minimum(v[i], v[j]), jnp.maximum(v[i], v[j])
            elif kind == 'min':
                v[i] = jnp.minimum(v[i], v[j])
            else:
                v[j] = jnp.maximum(v[i], v[j])
        return v[12]

    def it_body(_, x):
        return jnp.where(mask3, x, median25(x))

    inp = lax.fori_loop(0, 10, it_body, img)

    anyv = ((inp[0] > 0) | (inp[1] > 0) | (inp[2] > 0)).astype(jnp.float32)
    zc = jnp.zeros((1, W), jnp.float32)
    zr = jnp.zeros((H + 2, 1), jnp.float32)
    p = jnp.concatenate([zc, anyv, zc], axis=0)
    p = jnp.concatenate([zr, p, zr], axis=1)
    neigh = p[0:H, 0:W]
    for di in range(3):
        for dj in range(3):
            if di == 0 and dj == 0:
                continue
            neigh = neigh + p[di:di + H, dj:dj + W]
    upd = (neigh >= 6.0).astype(jnp.float32)
    out_ref[0] = inp * upd[None]


def _median_call(img, interpret=False):
    return pl.pallas_call(
        _median_body,
        grid=(BQ,),
        in_specs=[pl.BlockSpec((1, 3, H, W), lambda b: (b, 0, 0, 0))],
        out_specs=pl.BlockSpec((1, 3, H, W), lambda b: (b, 0, 0, 0)),
        out_shape=jax.ShapeDtypeStruct((BQ, 3, H, W), jnp.float32),
        interpret=interpret,
    )(img)


def kernel(cloud, rgb_vec, K, T):
    Tinv = jnp.linalg.inv(T)
    Kpad = jnp.pad(K, ((0, 0), (0, 1), (0, 1)))
    mats = jnp.concatenate([Tinv, Kpad], axis=1)
    cloud4 = cloud.reshape(BQ, 4, NRESH, 128)
    sk, fk = _proj_call(mats, cloud4)
    planes = _scatter_k(sk.reshape(PN), fk.reshape(PN))
    img_flat = _winner_k(planes, rgb_vec.reshape(PN * 3))
    return _median_call(img_flat.reshape(BQ, 3, H, W))


# speedup vs baseline: 32.5193x; 32.5193x over previous
import functools

import jax
import jax.numpy as jnp
from jax import lax
from jax.experimental import pallas as pl
from jax.experimental.pallas import tpu as pltpu
from jax.experimental.pallas import tpu_sc as plsc

H = 192
W = 256
BQ = 2
NPTS = 262144
SIZE = BQ * H * W
PN = BQ * NPTS
POSMASK = (1 << 19) - 1
I32MAX = 2**31 - 1

NC, NS = 2, 16
NW = NC * NS
PTS_PER_W = PN // NW
CHUNK = 4096
SLICE_M = SIZE // NS
SLICE_O = SIZE // NW
NROW = SLICE_O // 128

NRESH = 2048
RB = 64


def _batcher_pairs(n):
    p = 1
    while p < n:
        p *= 2
    pairs = []

    def merge(lo, hi, r):
        step = r * 2
        if step < hi - lo:
            merge(lo, hi, step)
            merge(lo + r, hi, step)
            for i in range(lo + r, hi - r, step):
                pairs.append((i, i + r))
        else:
            pairs.append((lo, lo + r))

    def sort(lo, hi):
        if (hi - lo) >= 1:
            mid = lo + ((hi - lo) // 2)
            sort(lo, mid)
            sort(mid + 1, hi)
            merge(lo, hi, 1)

    sort(0, p - 1)
    return [(i, j) for (i, j) in pairs if i < n and j < n]


def _prune(pairs, out_idx):
    needed = {out_idx}
    ops = []
    for (i, j) in reversed(pairs):
        ni, nj = i in needed, j in needed
        if not ni and not nj:
            continue
        ops.append(('ce' if (ni and nj) else ('min' if ni else 'max'), i, j))
        needed.add(i)
        needed.add(j)
    ops.reverse()
    return ops


_MED_OPS = _prune(_batcher_pairs(25), 12)


def _proj_body(mats_ref, cloud_ref, sk_ref, fk_ref):
    b = pl.program_id(0)
    j = pl.program_id(1)
    x = cloud_ref[0, 0]
    y = cloud_ref[0, 1]
    z = cloud_ref[0, 2]

    def m(i, k):
        return mats_ref[0, i, k]

    def rb(t):
        return t.astype(jnp.bfloat16).astype(jnp.float32)

    xb, yb, zb = rb(x), rb(y), rb(z)
    cam0 = xb * m(0, 0) + yb * m(0, 1) + zb * m(0, 2) + m(0, 3)
    cam1 = xb * m(1, 0) + yb * m(1, 1) + zb * m(1, 2) + m(1, 3)
    cam2 = xb * m(2, 0) + yb * m(2, 1) + zb * m(2, 2) + m(2, 3)
    c0, c1, c2 = rb(cam0), rb(cam1), rb(cam2)
    p0 = c0 * m(4, 0) + c1 * m(4, 1) + c2 * m(4, 2)
    p1 = c0 * m(5, 0) + c1 * m(5, 1) + c2 * m(5, 2)
    p2 = c0 * m(6, 0) + c1 * m(6, 1) + c2 * m(6, 2)
    u = jnp.clip((p0 / p2).astype(jnp.int32), 0, W - 1)
    v = jnp.clip((p1 / p2).astype(jnp.int32), 0, H - 1)
    pix = b * (H * W) + v * W + u
    di = cam2.astype(jnp.int32)
    row = j * RB + lax.broadcasted_iota(jnp.int32, (RB, 128), 0)
    col = lax.broadcasted_iota(jnp.int32, (RB, 128), 1)
    n = row * 128 + col
    posf = (PN - 1) - (b * NPTS + n)
    fk_ref[0] = jnp.clip(di, 0, 4094) * 524288 + posf
    lane_posf = 15 - (col % 16)
    sk_ref[0] = pix * 512 + jnp.clip(di, 0, 31) * 16 + lane_posf


def _proj_call(mats, cloud4, interpret=False):
    return pl.pallas_call(
        _proj_body,
        grid=(BQ, NRESH // RB),
        in_specs=[
            pl.BlockSpec((1, 8, 4), lambda b, j: (b, 0, 0),
                         memory_space=pltpu.SMEM),
            pl.BlockSpec((1, 4, RB, 128), lambda b, j: (b, 0, j, 0)),
        ],
        out_specs=[
            pl.BlockSpec((1, RB, 128), lambda b, j: (b, j, 0)),
            pl.BlockSpec((1, RB, 128), lambda b, j: (b, j, 0)),
        ],
        out_shape=[
            jax.ShapeDtypeStruct((BQ, NRESH, 128), jnp.int32),
            jax.ShapeDtypeStruct((BQ, NRESH, 128), jnp.int32),
        ],
        interpret=interpret,
    )(mats, cloud4)


@functools.cache
def _build_scatter_k():
    mesh = plsc.VectorSubcoreMesh(
        core_axis_name="c", subcore_axis_name="s",
        num_cores=NC, num_subcores=NS)
    return functools.partial(
        pl.kernel,
        out_type=jax.ShapeDtypeStruct((NW, SIZE), jnp.int32),
        mesh=mesh,
        compiler_params=pltpu.CompilerParams(needs_layout_passes=False),
        scratch_types=[
            pltpu.VMEM((SIZE,), jnp.int32),
            pltpu.VMEM((CHUNK,), jnp.int32),
            pltpu.VMEM((CHUNK,), jnp.int32),
            pltpu.VMEM((16,), jnp.int32),
        ],
    )(_scatter_body)


def _scatter_body(sk_hbm, fk_hbm, out_hbm, zbuf, ssk, sfk, svec):
    cid = lax.axis_index("c")
    sid = lax.axis_index("s")
    wid = cid * NS + sid
    iota = lax.broadcasted_iota(jnp.int32, (16,), 0)
    idxm1 = jnp.maximum(iota - 1, 0)
    maxv = jnp.full((16,), I32MAX, jnp.int32)

    def init_body(i, _):
        zbuf[pl.ds(i * 16, 16)] = maxv
        return 0

    lax.fori_loop(0, SIZE // 16, init_body, 0)

    base = wid * PTS_PER_W
    for ch in range(PTS_PER_W // CHUNK):
        off = base + ch * CHUNK
        pltpu.sync_copy(sk_hbm.at[pl.ds(off, CHUNK)], ssk)
        pltpu.sync_copy(fk_hbm.at[pl.ds(off, CHUNK)], sfk)

        def pt_body(i, _):
            sk = ssk[pl.ds(i * 16, 16)]
            fk = sfk[pl.ds(i * 16, 16)]
            k_s, v_s = plsc.sort_key_val(sk, fk)
            svec[...] = k_s
            prev = plsc.load_gather(svec, [idxm1])
            spix = lax.shift_right_logical(k_s, 9)
            ppix = lax.shift_right_logical(prev, 9)
            first = (spix != ppix) | (iota == 0)
            cur = plsc.load_gather(zbuf, [spix])
            newv = jnp.minimum(cur, v_s)
            plsc.store_scatter(zbuf, [spix], newv, mask=first)
            return 0

        lax.fori_loop(0, CHUNK // 16, pt_body, 0)

    pltpu.sync_copy(zbuf, out_hbm.at[wid])


@functools.cache
def _build_winner_k():
    mesh = plsc.VectorSubcoreMesh(
        core_axis_name="c", subcore_axis_name="s",
        num_cores=NC, num_subcores=NS)
    return functools.partial(
        pl.kernel,
        out_type=jax.ShapeDtypeStruct((BQ * 3 * H * W,), jnp.float32),
        mesh=mesh,
        compiler_params=pltpu.CompilerParams(needs_layout_passes=False),
        scratch_types=[
            pltpu.VMEM((SLICE_O,), jnp.int32),
            pltpu.VMEM((SLICE_O,), jnp.int32),
            pltpu.VMEM((3 * SLICE_O,), jnp.int32),
            pltpu.VMEM((3 * SLICE_O,), jnp.float32),
            pltpu.VMEM((3 * SLICE_O,), jnp.float32),
            pltpu.SemaphoreType.DMA,
        ],
    )(_winner_body)


def _winner_body(p_hbm, rgb_hbm, img_hbm, k0, k1, idxr, gat, ost, sem):
    cid = lax.axis_index("c")
    sid = lax.axis_index("s")
    wid = cid * NS + sid
    pb = wid * SLICE_O
    b = wid // NS
    pltpu.sync_copy(p_hbm.at[0, pl.ds(pb, SLICE_O)], k0)
    for j in range(1, NW):
        pltpu.sync_copy(p_hbm.at[j, pl.ds(pb, SLICE_O)], k1)

        def merge_body(i, _):
            k0[pl.ds(i * 16, 16)] = jnp.minimum(
                k0[pl.ds(i * 16, 16)], k1[pl.ds(i * 16, 16)])
            return 0

        lax.fori_loop(0, SLICE_O // 16, merge_body, 0)
    iota = lax.broadcasted_iota(jnp.int32, (16,), 0)

    def idx_body(i, _):
        a = k0[pl.ds(i * 16, 16)]
        valid = a != I32MAX
        posf = a & POSMASK
        n = (PN - 1) - posf - b * NPTS
        spread = (pb + i * 16 + iota) & 1023
        base_i = b * (3 * NPTS) + n
        for c in range(3):
            idxr[pl.ds(c * SLICE_O + i * 16, 16)] = jnp.where(
                valid, base_i + c * NPTS, spread)
        return 0

    lax.fori_loop(0, SLICE_O // 16, idx_body, 0)

    for c in range(3):
        copies = [
            pltpu.async_copy(
                rgb_hbm.at[idxr.at[pl.ds((c * NROW + r) * 128, 128)]],
                gat.at[pl.ds((c * NROW + r) * 128, 128)], sem)
            for r in range(NROW)
        ]
        for cp in copies:
            cp.wait()

    def out_body(i, _):
        a = k0[pl.ds(i * 16, 16)]
        valid = a != I32MAX
        for c in range(3):
            g = gat[pl.ds(c * SLICE_O + i * 16, 16)]
            ost[pl.ds(c * SLICE_O + i * 16, 16)] = jnp.where(
                valid, g, -0.001)
        return 0

    lax.fori_loop(0, SLICE_O // 16, out_body, 0)

    pwb = pb - b * (H * W)
    for c in range(3):
        dst = b * (3 * H * W) + c * (H * W) + pwb
        pltpu.sync_copy(ost.at[pl.ds(c * SLICE_O, SLICE_O)],
                        img_hbm.at[pl.ds(dst, SLICE_O)])


def _median_body(img_ref, out_ref):
    img = img_ref[0]
    mask3 = jnp.broadcast_to(img[0:1] > 0, (3, H, W))

    def pad_reflect(x):
        x = jnp.concatenate(
            [x[:, 2:3], x[:, 1:2], x, x[:, H - 2:H - 1], x[:, H - 3:H - 2]],
            axis=1)
        x = jnp.concatenate(
            [x[:, :, 2:3], x[:, :, 1:2], x,
             x[:, :, W - 2:W - 1], x[:, :, W - 3:W - 2]],
            axis=2)
        return x

    def median25(x):
        p = pad_reflect(x)
        v = [p[:, di:di + H, dj:dj + W] for di in range(5) for dj in range(5)]
        for kind, i, j in _MED_OPS:
            if kind == 'ce':
                v[i], v[j] = jnp.minimum(v[i], v[j]), jnp.maximum(v[i], v[j])
            elif kind == 'min':
                v[i] = jnp.minimum(v[i], v[j])
            else:
                v[j] = jnp.maximum(v[i], v[j])
        return v[12]

    def it_body(_, x):
        return jnp.where(mask3, x, median25(x))

    inp = lax.fori_loop(0, 10, it_body, img)

    anyv = ((inp[0] > 0) | (inp[1] > 0) | (inp[2] > 0)).astype(jnp.float32)
    zc = jnp.zeros((1, W), jnp.float32)
    zr = jnp.zeros((H + 2, 1), jnp.float32)
    p = jnp.concatenate([zc, anyv, zc], axis=0)
    p = jnp.concatenate([zr, p, zr], axis=1)
    neigh = p[0:H, 0:W]
    for di in range(3):
        for dj in range(3):
            if di == 0 and dj == 0:
                continue
            neigh = neigh + p[di:di + H, dj:dj + W]
    upd = (neigh >= 6.0).astype(jnp.float32)
    out_ref[0] = inp * upd[None]


def _median_call(img, interpret=False):
    return pl.pallas_call(
        _median_body,
        grid=(BQ,),
        in_specs=[pl.BlockSpec((1, 3, H, W), lambda b: (b, 0, 0, 0))],
        out_specs=pl.BlockSpec((1, 3, H, W), lambda b: (b, 0, 0, 0)),
        out_shape=jax.ShapeDtypeStruct((BQ, 3, H, W), jnp.float32),
        interpret=interpret,
    )(img)


def kernel(cloud, rgb_vec, K, T):
    Tinv = jnp.linalg.inv(T)
    Kpad = jnp.pad(K, ((0, 0), (0, 1), (0, 1)))
    mats = jnp.concatenate([Tinv, Kpad], axis=1)
    mats = mats.astype(jnp.bfloat16).astype(jnp.float32)
    cloud4 = cloud.reshape(BQ, 4, NRESH, 128)
    sk, fk = _proj_call(mats, cloud4)
    planes = _build_scatter_k()(sk.reshape(PN), fk.reshape(PN))
    img_flat = _build_winner_k()(planes, rgb_vec.reshape(PN * 3))
    return _median_call(img_flat.reshape(BQ, 3, H, W))


# speedup vs baseline: 36.7181x; 1.1291x over previous
import functools

import jax
import jax.numpy as jnp
from jax import lax
from jax.experimental import pallas as pl
from jax.experimental.pallas import tpu as pltpu
from jax.experimental.pallas import tpu_sc as plsc

H = 192
W = 256
BQ = 2
NPTS = 262144
SIZE = BQ * H * W
PN = BQ * NPTS
POSMASK = (1 << 19) - 1
I32MAX = 2**31 - 1

NC, NS = 2, 16
NW = NC * NS
PTS_PER_W = PN // NW
CHUNK = 4096
SLICE_M = SIZE // NS
SLICE_O = SIZE // NW
NROW = SLICE_O // 128

NRESH = 2048
RB = 64


def _batcher_pairs(n):
    p = 1
    while p < n:
        p *= 2
    pairs = []

    def merge(lo, hi, r):
        step = r * 2
        if step < hi - lo:
            merge(lo, hi, step)
            merge(lo + r, hi, step)
            for i in range(lo + r, hi - r, step):
                pairs.append((i, i + r))
        else:
            pairs.append((lo, lo + r))

    def sort(lo, hi):
        if (hi - lo) >= 1:
            mid = lo + ((hi - lo) // 2)
            sort(lo, mid)
            sort(mid + 1, hi)
            merge(lo, hi, 1)

    sort(0, p - 1)
    return [(i, j) for (i, j) in pairs if i < n and j < n]


def _prune(pairs, out_idx):
    needed = {out_idx}
    ops = []
    for (i, j) in reversed(pairs):
        ni, nj = i in needed, j in needed
        if not ni and not nj:
            continue
        ops.append(('ce' if (ni and nj) else ('min' if ni else 'max'), i, j))
        needed.add(i)
        needed.add(j)
    ops.reverse()
    return ops


_MED_OPS = _prune(_batcher_pairs(25), 12)


def _proj_body(mats_ref, cloud_ref, sk_ref, fk_ref):
    b = pl.program_id(0)
    j = pl.program_id(1)
    x = cloud_ref[0, 0]
    y = cloud_ref[0, 1]
    z = cloud_ref[0, 2]

    def m(i, k):
        return mats_ref[0, i, k]

    def rb(t):
        return t.astype(jnp.bfloat16).astype(jnp.float32)

    xb, yb, zb = rb(x), rb(y), rb(z)
    cam0 = xb * m(0, 0) + yb * m(0, 1) + zb * m(0, 2) + m(0, 3)
    cam1 = xb * m(1, 0) + yb * m(1, 1) + zb * m(1, 2) + m(1, 3)
    cam2 = xb * m(2, 0) + yb * m(2, 1) + zb * m(2, 2) + m(2, 3)
    c0, c1, c2 = rb(cam0), rb(cam1), rb(cam2)
    p0 = c0 * m(4, 0) + c1 * m(4, 1) + c2 * m(4, 2)
    p1 = c0 * m(5, 0) + c1 * m(5, 1) + c2 * m(5, 2)
    p2 = c0 * m(6, 0) + c1 * m(6, 1) + c2 * m(6, 2)
    u = jnp.clip((p0 / p2).astype(jnp.int32), 0, W - 1)
    v = jnp.clip((p1 / p2).astype(jnp.int32), 0, H - 1)
    pix = b * (H * W) + v * W + u
    di = cam2.astype(jnp.int32)
    row = j * RB + lax.broadcasted_iota(jnp.int32, (RB, 128), 0)
    col = lax.broadcasted_iota(jnp.int32, (RB, 128), 1)
    n = row * 128 + col
    posf = (PN - 1) - (b * NPTS + n)
    fk_ref[0] = jnp.clip(di, 0, 4094) * 524288 + posf
    lane_posf = 15 - (col % 16)
    sk_ref[0] = pix * 512 + jnp.clip(di, 0, 31) * 16 + lane_posf


def _proj_call(mats, cloud4, interpret=False):
    return pl.pallas_call(
        _proj_body,
        grid=(BQ, NRESH // RB),
        in_specs=[
            pl.BlockSpec((1, 8, 4), lambda b, j: (b, 0, 0),
                         memory_space=pltpu.SMEM),
            pl.BlockSpec((1, 4, RB, 128), lambda b, j: (b, 0, j, 0)),
        ],
        out_specs=[
            pl.BlockSpec((1, RB, 128), lambda b, j: (b, j, 0)),
            pl.BlockSpec((1, RB, 128), lambda b, j: (b, j, 0)),
        ],
        out_shape=[
            jax.ShapeDtypeStruct((BQ, NRESH, 128), jnp.int32),
            jax.ShapeDtypeStruct((BQ, NRESH, 128), jnp.int32),
        ],
        interpret=interpret,
    )(mats, cloud4)


@functools.cache
def _build_scatter_k():
    mesh = plsc.VectorSubcoreMesh(
        core_axis_name="c", subcore_axis_name="s",
        num_cores=NC, num_subcores=NS)
    return functools.partial(
        pl.kernel,
        out_type=jax.ShapeDtypeStruct((NW, SIZE), jnp.int32),
        mesh=mesh,
        compiler_params=pltpu.CompilerParams(needs_layout_passes=False),
        scratch_types=[
            pltpu.VMEM((SIZE,), jnp.int32),
            pltpu.VMEM((CHUNK,), jnp.int32),
            pltpu.VMEM((CHUNK,), jnp.int32),
            pltpu.VMEM((16,), jnp.int32),
            pltpu.VMEM((16,), jnp.int32),
        ],
    )(_scatter_body)


def _scatter_body(sk_hbm, fk_hbm, out_hbm, zbuf, ssk, sfk, svec0, svec1):
    cid = lax.axis_index("c")
    sid = lax.axis_index("s")
    wid = cid * NS + sid
    iota = lax.broadcasted_iota(jnp.int32, (16,), 0)
    idxm1 = jnp.maximum(iota - 1, 0)
    maxv = jnp.full((16,), I32MAX, jnp.int32)
    svecs = (svec0, svec1)

    def init_body(i, _):
        for u in range(8):
            zbuf[pl.ds((i * 8 + u) * 16, 16)] = maxv
        return 0

    lax.fori_loop(0, SIZE // 128, init_body, 0)

    base = wid * PTS_PER_W
    for ch in range(PTS_PER_W // CHUNK):
        off = base + ch * CHUNK
        pltpu.sync_copy(sk_hbm.at[pl.ds(off, CHUNK)], ssk)
        pltpu.sync_copy(fk_hbm.at[pl.ds(off, CHUNK)], sfk)

        def pt_body(i, _):
            for u in range(2):
                o = (i * 2 + u) * 16
                sk = ssk[pl.ds(o, 16)]
                fk = sfk[pl.ds(o, 16)]
                k_s, v_s = plsc.sort_key_val(sk, fk)
                svecs[u][...] = k_s
                prev = plsc.load_gather(svecs[u], [idxm1])
                spix = lax.shift_right_logical(k_s, 9)
                ppix = lax.shift_right_logical(prev, 9)
                first = (spix != ppix) | (iota == 0)
                cur = plsc.load_gather(zbuf, [spix])
                newv = jnp.minimum(cur, v_s)
                plsc.store_scatter(zbuf, [spix], newv, mask=first)
            return 0

        lax.fori_loop(0, CHUNK // 32, pt_body, 0)

    pltpu.sync_copy(zbuf, out_hbm.at[wid])


@functools.cache
def _build_winner_k():
    mesh = plsc.VectorSubcoreMesh(
        core_axis_name="c", subcore_axis_name="s",
        num_cores=NC, num_subcores=NS)
    return functools.partial(
        pl.kernel,
        out_type=jax.ShapeDtypeStruct((BQ * 3 * H * W,), jnp.float32),
        mesh=mesh,
        compiler_params=pltpu.CompilerParams(needs_layout_passes=False),
        scratch_types=[
            pltpu.VMEM((SLICE_O,), jnp.int32),
            pltpu.VMEM((SLICE_O,), jnp.int32),
            pltpu.VMEM((SLICE_O,), jnp.int32),
            pltpu.VMEM((SLICE_O,), jnp.int32),
            pltpu.VMEM((SLICE_O,), jnp.int32),
            pltpu.VMEM((SLICE_O,), jnp.int32),
            pltpu.VMEM((3 * SLICE_O,), jnp.float32),
            pltpu.VMEM((3 * SLICE_O,), jnp.float32),
            pltpu.SemaphoreType.DMA,
            pltpu.SemaphoreType.DMA,
            pltpu.SemaphoreType.DMA,
        ],
    )(_winner_body)


def _winner_body(p_hbm, rgb_hbm, img_hbm, k0, kA, kB, ix0, ix1, ix2, gat,
                 ost, semA, semB, semG):
    cid = lax.axis_index("c")
    sid = lax.axis_index("s")
    wid = cid * NS + sid
    pb = wid * SLICE_O
    b = wid // NS
    pltpu.sync_copy(p_hbm.at[0, pl.ds(pb, SLICE_O)], k0)
    bufs = (kA, kB)
    sems = (semA, semB)
    cps = [None, None]
    cps[1] = pltpu.async_copy(p_hbm.at[1, pl.ds(pb, SLICE_O)], bufs[1],
                              sems[1])
    for j in range(1, NW):
        if j + 1 < NW:
            nb = (j + 1) % 2
            cps[nb] = pltpu.async_copy(
                p_hbm.at[j + 1, pl.ds(pb, SLICE_O)], bufs[nb], sems[nb])
        cps[j % 2].wait()
        buf = bufs[j % 2]

        def merge_body(i, _):
            for u in range(4):
                o = (i * 4 + u) * 16
                k0[pl.ds(o, 16)] = jnp.minimum(
                    k0[pl.ds(o, 16)], buf[pl.ds(o, 16)])
            return 0

        lax.fori_loop(0, SLICE_O // 64, merge_body, 0)
    iota = lax.broadcasted_iota(jnp.int32, (16,), 0)
    ixs = (ix0, ix1, ix2)

    def idx_body(i, _):
        a = k0[pl.ds(i * 16, 16)]
        valid = a != I32MAX
        posf = a & POSMASK
        n = (PN - 1) - posf - b * NPTS
        spread = (pb + i * 16 + iota) & 1023
        base_i = b * (3 * NPTS) + n
        for c in range(3):
            ixs[c][pl.ds(i * 16, 16)] = jnp.where(
                valid, base_i + c * NPTS, spread)
        return 0

    lax.fori_loop(0, SLICE_O // 16, idx_body, 0)

    copies = [
        pltpu.async_copy(rgb_hbm.at[ixs[c]],
                         gat.at[pl.ds(c * SLICE_O, SLICE_O)], semG)
        for c in range(3)
    ]
    for cp in copies:
        cp.wait()

    def out_body(i, _):
        a = k0[pl.ds(i * 16, 16)]
        valid = a != I32MAX
        for c in range(3):
            g = gat[pl.ds(c * SLICE_O + i * 16, 16)]
            ost[pl.ds(c * SLICE_O + i * 16, 16)] = jnp.where(
                valid, g, -0.001)
        return 0

    lax.fori_loop(0, SLICE_O // 16, out_body, 0)

    pwb = pb - b * (H * W)
    for c in range(3):
        dst = b * (3 * H * W) + c * (H * W) + pwb
        pltpu.sync_copy(ost.at[pl.ds(c * SLICE_O, SLICE_O)],
                        img_hbm.at[pl.ds(dst, SLICE_O)])


def _median_body(img_ref, out_ref):
    img = img_ref[0]
    mask3 = jnp.broadcast_to(img[0:1] > 0, (3, H, W))

    def pad_reflect(x):
        x = jnp.concatenate(
            [x[:, 2:3], x[:, 1:2], x, x[:, H - 2:H - 1], x[:, H - 3:H - 2]],
            axis=1)
        x = jnp.concatenate(
            [x[:, :, 2:3], x[:, :, 1:2], x,
             x[:, :, W - 2:W - 1], x[:, :, W - 3:W - 2]],
            axis=2)
        return x

    def median25(x):
        p = pad_reflect(x)
        v = [p[:, di:di + H, dj:dj + W] for di in range(5) for dj in range(5)]
        for kind, i, j in _MED_OPS:
            if kind == 'ce':
                v[i], v[j] = jnp.minimum(v[i], v[j]), jnp.maximum(v[i], v[j])
            elif kind == 'min':
                v[i] = jnp.minimum(v[i], v[j])
            else:
                v[j] = jnp.maximum(v[i], v[j])
        return v[12]

    def it_body(_, x):
        return jnp.where(mask3, x, median25(x))

    inp = lax.fori_loop(0, 10, it_body, img)

    anyv = ((inp[0] > 0) | (inp[1] > 0) | (inp[2] > 0)).astype(jnp.float32)
    zc = jnp.zeros((1, W), jnp.float32)
    zr = jnp.zeros((H + 2, 1), jnp.float32)
    p = jnp.concatenate([zc, anyv, zc], axis=0)
    p = jnp.concatenate([zr, p, zr], axis=1)
    neigh = p[0:H, 0:W]
    for di in range(3):
        for dj in range(3):
            if di == 0 and dj == 0:
                continue
            neigh = neigh + p[di:di + H, dj:dj + W]
    upd = (neigh >= 6.0).astype(jnp.float32)
    out_ref[0] = inp * upd[None]


def _median_call(img, interpret=False):
    return pl.pallas_call(
        _median_body,
        grid=(BQ,),
        in_specs=[pl.BlockSpec((1, 3, H, W), lambda b: (b, 0, 0, 0))],
        out_specs=pl.BlockSpec((1, 3, H, W), lambda b: (b, 0, 0, 0)),
        out_shape=jax.ShapeDtypeStruct((BQ, 3, H, W), jnp.float32),
        interpret=interpret,
    )(img)


def kernel(cloud, rgb_vec, K, T):
    Tinv = jnp.linalg.inv(T)
    Kpad = jnp.pad(K, ((0, 0), (0, 1), (0, 1)))
    mats = jnp.concatenate([Tinv, Kpad], axis=1)
    mats = mats.astype(jnp.bfloat16).astype(jnp.float32)
    cloud4 = cloud.reshape(BQ, 4, NRESH, 128)
    sk, fk = _proj_call(mats, cloud4)
    planes = _build_scatter_k()(sk.reshape(PN), fk.reshape(PN))
    img_flat = _build_winner_k()(planes, rgb_vec.reshape(PN * 3))
    return _median_call(img_flat.reshape(BQ, 3, H, W))


# speedup vs baseline: 48.7809x; 1.3285x over previous
import functools

import jax
import jax.numpy as jnp
from jax import lax
from jax.experimental import pallas as pl
from jax.experimental.pallas import tpu as pltpu
from jax.experimental.pallas import tpu_sc as plsc

H = 192
W = 256
BQ = 2
NPTS = 262144
SIZE = BQ * H * W
PN = BQ * NPTS
POSMASK = (1 << 19) - 1
I32MAX = 2**31 - 1

NC, NS = 2, 16
NW = NC * NS
PTS_PER_W = PN // NW
CHUNK = 4096
SLICE_M = SIZE // NS
SLICE_O = SIZE // NW
NROW = SLICE_O // 128

NRESH = 2048
RB = 64


def _batcher_pairs(n):
    p = 1
    while p < n:
        p *= 2
    pairs = []

    def merge(lo, hi, r):
        step = r * 2
        if step < hi - lo:
            merge(lo, hi, step)
            merge(lo + r, hi, step)
            for i in range(lo + r, hi - r, step):
                pairs.append((i, i + r))
        else:
            pairs.append((lo, lo + r))

    def sort(lo, hi):
        if (hi - lo) >= 1:
            mid = lo + ((hi - lo) // 2)
            sort(lo, mid)
            sort(mid + 1, hi)
            merge(lo, hi, 1)

    sort(0, p - 1)
    return [(i, j) for (i, j) in pairs if i < n and j < n]


def _prune(pairs, out_idx):
    needed = {out_idx}
    ops = []
    for (i, j) in reversed(pairs):
        ni, nj = i in needed, j in needed
        if not ni and not nj:
            continue
        ops.append(('ce' if (ni and nj) else ('min' if ni else 'max'), i, j))
        needed.add(i)
        needed.add(j)
    ops.reverse()
    return ops


_MED_OPS = _prune(_batcher_pairs(25), 12)


def _proj_body(mats_ref, cloud_ref, sk_ref, fk_ref):
    b = pl.program_id(0)
    j = pl.program_id(1)
    x = cloud_ref[0, 0]
    y = cloud_ref[0, 1]
    z = cloud_ref[0, 2]

    def m(i, k):
        return mats_ref[0, i, k]

    def rb(t):
        return t.astype(jnp.bfloat16).astype(jnp.float32)

    xb, yb, zb = rb(x), rb(y), rb(z)
    cam0 = xb * m(0, 0) + yb * m(0, 1) + zb * m(0, 2) + m(0, 3)
    cam1 = xb * m(1, 0) + yb * m(1, 1) + zb * m(1, 2) + m(1, 3)
    cam2 = xb * m(2, 0) + yb * m(2, 1) + zb * m(2, 2) + m(2, 3)
    c0, c1, c2 = rb(cam0), rb(cam1), rb(cam2)
    p0 = c0 * m(4, 0) + c1 * m(4, 1) + c2 * m(4, 2)
    p1 = c0 * m(5, 0) + c1 * m(5, 1) + c2 * m(5, 2)
    p2 = c0 * m(6, 0) + c1 * m(6, 1) + c2 * m(6, 2)
    u = jnp.clip((p0 / p2).astype(jnp.int32), 0, W - 1)
    v = jnp.clip((p1 / p2).astype(jnp.int32), 0, H - 1)
    pix = b * (H * W) + v * W + u
    di = cam2.astype(jnp.int32)
    row = j * RB + lax.broadcasted_iota(jnp.int32, (RB, 128), 0)
    col = lax.broadcasted_iota(jnp.int32, (RB, 128), 1)
    n = row * 128 + col
    posf = (PN - 1) - (b * NPTS + n)
    fk_ref[0] = jnp.clip(di, 0, 4094) * 524288 + posf
    lane_posf = 15 - (col % 16)
    sk_ref[0] = pix * 512 + jnp.clip(di, 0, 31) * 16 + lane_posf


def _proj_call(mats, cloud4, interpret=False):
    return pl.pallas_call(
        _proj_body,
        grid=(BQ, NRESH // RB),
        in_specs=[
            pl.BlockSpec((1, 8, 4), lambda b, j: (b, 0, 0),
                         memory_space=pltpu.SMEM),
            pl.BlockSpec((1, 4, RB, 128), lambda b, j: (b, 0, j, 0)),
        ],
        out_specs=[
            pl.BlockSpec((1, RB, 128), lambda b, j: (b, j, 0)),
            pl.BlockSpec((1, RB, 128), lambda b, j: (b, j, 0)),
        ],
        out_shape=[
            jax.ShapeDtypeStruct((BQ, NRESH, 128), jnp.int32),
            jax.ShapeDtypeStruct((BQ, NRESH, 128), jnp.int32),
        ],
        interpret=interpret,
    )(mats, cloud4)


@functools.cache
def _build_scatter_k():
    mesh = plsc.VectorSubcoreMesh(
        core_axis_name="c", subcore_axis_name="s",
        num_cores=NC, num_subcores=NS)
    return functools.partial(
        pl.kernel,
        out_type=jax.ShapeDtypeStruct((NW, SIZE), jnp.int32),
        mesh=mesh,
        compiler_params=pltpu.CompilerParams(needs_layout_passes=False),
        scratch_types=[
            pltpu.VMEM((SIZE,), jnp.int32),
            pltpu.VMEM((CHUNK,), jnp.int32),
            pltpu.VMEM((CHUNK,), jnp.int32),
            pltpu.VMEM((16,), jnp.int32),
            pltpu.VMEM((16,), jnp.int32),
        ],
    )(_scatter_body)


def _scatter_body(sk_hbm, fk_hbm, out_hbm, zbuf, ssk, sfk, svec0, svec1):
    cid = lax.axis_index("c")
    sid = lax.axis_index("s")
    wid = cid * NS + sid
    iota = lax.broadcasted_iota(jnp.int32, (16,), 0)
    idxm1 = jnp.maximum(iota - 1, 0)
    maxv = jnp.full((16,), I32MAX, jnp.int32)
    svecs = (svec0, svec1)

    def init_body(i, _):
        for u in range(8):
            zbuf[pl.ds((i * 8 + u) * 16, 16)] = maxv
        return 0

    lax.fori_loop(0, SIZE // 128, init_body, 0)

    base = wid * PTS_PER_W
    for ch in range(PTS_PER_W // CHUNK):
        off = base + ch * CHUNK
        pltpu.sync_copy(sk_hbm.at[pl.ds(off, CHUNK)], ssk)
        pltpu.sync_copy(fk_hbm.at[pl.ds(off, CHUNK)], sfk)

        def pt_body(i, _):
            for u in range(2):
                o = (i * 2 + u) * 16
                sk = ssk[pl.ds(o, 16)]
                fk = sfk[pl.ds(o, 16)]
                k_s, v_s = plsc.sort_key_val(sk, fk)
                svecs[u][...] = k_s
                prev = plsc.load_gather(svecs[u], [idxm1])
                spix = lax.shift_right_logical(k_s, 9)
                ppix = lax.shift_right_logical(prev, 9)
                first = (spix != ppix) | (iota == 0)
                cur = plsc.load_gather(zbuf, [spix])
                newv = jnp.minimum(cur, v_s)
                plsc.store_scatter(zbuf, [spix], newv, mask=first)
            return 0

        lax.fori_loop(0, CHUNK // 32, pt_body, 0)

    pltpu.sync_copy(zbuf, out_hbm.at[wid])


@functools.cache
def _build_winner_k():
    mesh = plsc.VectorSubcoreMesh(
        core_axis_name="c", subcore_axis_name="s",
        num_cores=NC, num_subcores=NS)
    return functools.partial(
        pl.kernel,
        out_type=jax.ShapeDtypeStruct((BQ * 3 * H * W,), jnp.float32),
        mesh=mesh,
        compiler_params=pltpu.CompilerParams(needs_layout_passes=False),
        scratch_types=[
            pltpu.VMEM((SLICE_O,), jnp.int32),
            pltpu.VMEM((SLICE_O,), jnp.int32),
            pltpu.VMEM((SLICE_O,), jnp.int32),
            pltpu.VMEM((SLICE_O,), jnp.int32),
            pltpu.VMEM((SLICE_O,), jnp.int32),
            pltpu.VMEM((SLICE_O,), jnp.int32),
            pltpu.VMEM((3 * SLICE_O,), jnp.float32),
            pltpu.VMEM((3 * SLICE_O,), jnp.float32),
            pltpu.SemaphoreType.DMA,
            pltpu.SemaphoreType.DMA,
            pltpu.SemaphoreType.DMA,
        ],
    )(_winner_body)


def _winner_body(p_hbm, rgb_hbm, img_hbm, k0, kA, kB, ix0, ix1, ix2, gat,
                 ost, semA, semB, semG):
    cid = lax.axis_index("c")
    sid = lax.axis_index("s")
    wid = cid * NS + sid
    pb = wid * SLICE_O
    b = wid // NS
    pltpu.sync_copy(p_hbm.at[0, pl.ds(pb, SLICE_O)], k0)
    bufs = (kA, kB)
    sems = (semA, semB)
    cps = [None, None]
    cps[1] = pltpu.async_copy(p_hbm.at[1, pl.ds(pb, SLICE_O)], bufs[1],
                              sems[1])
    for j in range(1, NW):
        if j + 1 < NW:
            nb = (j + 1) % 2
            cps[nb] = pltpu.async_copy(
                p_hbm.at[j + 1, pl.ds(pb, SLICE_O)], bufs[nb], sems[nb])
        cps[j % 2].wait()
        buf = bufs[j % 2]

        def merge_body(i, _):
            for u in range(4):
                o = (i * 4 + u) * 16
                k0[pl.ds(o, 16)] = jnp.minimum(
                    k0[pl.ds(o, 16)], buf[pl.ds(o, 16)])
            return 0

        lax.fori_loop(0, SLICE_O // 64, merge_body, 0)
    iota = lax.broadcasted_iota(jnp.int32, (16,), 0)
    ixs = (ix0, ix1, ix2)

    def idx_body(i, _):
        a = k0[pl.ds(i * 16, 16)]
        valid = a != I32MAX
        posf = a & POSMASK
        n = (PN - 1) - posf - b * NPTS
        spread = (pb + i * 16 + iota) & 1023
        base_i = b * (3 * NPTS) + n
        for c in range(3):
            ixs[c][pl.ds(i * 16, 16)] = jnp.where(
                valid, base_i + c * NPTS, spread)
        return 0

    lax.fori_loop(0, SLICE_O // 16, idx_body, 0)

    copies = [
        pltpu.async_copy(rgb_hbm.at[ixs[c]],
                         gat.at[pl.ds(c * SLICE_O, SLICE_O)], semG)
        for c in range(3)
    ]
    for cp in copies:
        cp.wait()

    def out_body(i, _):
        a = k0[pl.ds(i * 16, 16)]
        valid = a != I32MAX
        for c in range(3):
            g = gat[pl.ds(c * SLICE_O + i * 16, 16)]
            ost[pl.ds(c * SLICE_O + i * 16, 16)] = jnp.where(
                valid, g, -0.001)
        return 0

    lax.fori_loop(0, SLICE_O // 16, out_body, 0)

    pwb = pb - b * (H * W)
    for c in range(3):
        dst = b * (3 * H * W) + c * (H * W) + pwb
        pltpu.sync_copy(ost.at[pl.ds(c * SLICE_O, SLICE_O)],
                        img_hbm.at[pl.ds(dst, SLICE_O)])


def _median_body(img_ref, out_ref):
    img = img_ref[0]
    mask3 = jnp.broadcast_to(img[0:1] > 0, (3, H, W))

    def pad_reflect(x):
        x = jnp.concatenate(
            [x[:, 2:3], x[:, 1:2], x, x[:, H - 2:H - 1], x[:, H - 3:H - 2]],
            axis=1)
        x = jnp.concatenate(
            [x[:, :, 2:3], x[:, :, 1:2], x,
             x[:, :, W - 2:W - 1], x[:, :, W - 3:W - 2]],
            axis=2)
        return x

    def median25(x):
        p = pad_reflect(x).astype(jnp.bfloat16)
        v = [p[:, di:di + H, dj:dj + W] for di in range(5) for dj in range(5)]
        for kind, i, j in _MED_OPS:
            if kind == 'ce':
                v[i], v[j] = jnp.minimum(v[i], v[j]), jnp.maximum(v[i], v[j])
            elif kind == 'min':
                v[i] = jnp.minimum(v[i], v[j])
            else:
                v[j] = jnp.maximum(v[i], v[j])
        return v[12].astype(jnp.float32)

    def it_body(_, x):
        return jnp.where(mask3, x, median25(x))

    inp = lax.fori_loop(0, 10, it_body, img)

    anyv = ((inp[0] > 0) | (inp[1] > 0) | (inp[2] > 0)).astype(jnp.float32)
    zc = jnp.zeros((1, W), jnp.float32)
    zr = jnp.zeros((H + 2, 1), jnp.float32)
    p = jnp.concatenate([zc, anyv, zc], axis=0)
    p = jnp.concatenate([zr, p, zr], axis=1)
    neigh = p[0:H, 0:W]
    for di in range(3):
        for dj in range(3):
            if di == 0 and dj == 0:
                continue
            neigh = neigh + p[di:di + H, dj:dj + W]
    upd = (neigh >= 6.0).astype(jnp.float32)
    out_ref[0] = inp * upd[None]


def _median_call(img, interpret=False):
    return pl.pallas_call(
        _median_body,
        grid=(BQ,),
        in_specs=[pl.BlockSpec((1, 3, H, W), lambda b: (b, 0, 0, 0))],
        out_specs=pl.BlockSpec((1, 3, H, W), lambda b: (b, 0, 0, 0)),
        out_shape=jax.ShapeDtypeStruct((BQ, 3, H, W), jnp.float32),
        interpret=interpret,
    )(img)


def kernel(cloud, rgb_vec, K, T):
    Tinv = jnp.linalg.inv(T)
    Kpad = jnp.pad(K, ((0, 0), (0, 1), (0, 1)))
    mats = jnp.concatenate([Tinv, Kpad], axis=1)
    mats = mats.astype(jnp.bfloat16).astype(jnp.float32)
    cloud4 = cloud.reshape(BQ, 4, NRESH, 128)
    sk, fk = _proj_call(mats, cloud4)
    planes = _build_scatter_k()(sk.reshape(PN), fk.reshape(PN))
    img_flat = _build_winner_k()(planes, rgb_vec.reshape(PN * 3))
    return _median_call(img_flat.reshape(BQ, 3, H, W))


# speedup vs baseline: 50.7712x; 1.0408x over previous
import functools

import jax
import jax.numpy as jnp
from jax import lax
from jax.experimental import pallas as pl
from jax.experimental.pallas import tpu as pltpu
from jax.experimental.pallas import tpu_sc as plsc

H = 192
W = 256
BQ = 2
NPTS = 262144
SIZE = BQ * H * W
PN = BQ * NPTS
POSMASK = (1 << 19) - 1
I32MAX = 2**31 - 1

NC, NS = 2, 16
NW = NC * NS
PTS_PER_W = PN // NW
CHUNK = 4096
SLICE_M = SIZE // NS
SLICE_O = SIZE // NW
NROW = SLICE_O // 128

def _batcher_pairs(n):
    p = 1
    while p < n:
        p *= 2
    pairs = []

    def merge(lo, hi, r):
        step = r * 2
        if step < hi - lo:
            merge(lo, hi, step)
            merge(lo + r, hi, step)
            for i in range(lo + r, hi - r, step):
                pairs.append((i, i + r))
        else:
            pairs.append((lo, lo + r))

    def sort(lo, hi):
        if (hi - lo) >= 1:
            mid = lo + ((hi - lo) // 2)
            sort(lo, mid)
            sort(mid + 1, hi)
            merge(lo, hi, 1)

    sort(0, p - 1)
    return [(i, j) for (i, j) in pairs if i < n and j < n]


def _prune(pairs, out_idx):
    needed = {out_idx}
    ops = []
    for (i, j) in reversed(pairs):
        ni, nj = i in needed, j in needed
        if not ni and not nj:
            continue
        ops.append(('ce' if (ni and nj) else ('min' if ni else 'max'), i, j))
        needed.add(i)
        needed.add(j)
    ops.reverse()
    return ops


_MED_OPS = _prune(_batcher_pairs(25), 12)


@functools.cache
def _build_scatter_k():
    mesh = plsc.VectorSubcoreMesh(
        core_axis_name="c", subcore_axis_name="s",
        num_cores=NC, num_subcores=NS)
    return functools.partial(
        pl.kernel,
        out_type=jax.ShapeDtypeStruct((NW, SIZE), jnp.int32),
        mesh=mesh,
        compiler_params=pltpu.CompilerParams(needs_layout_passes=False),
        scratch_types=[
            pltpu.VMEM((SIZE,), jnp.int32),
            pltpu.VMEM((CHUNK,), jnp.float32),
            pltpu.VMEM((CHUNK,), jnp.float32),
            pltpu.VMEM((CHUNK,), jnp.float32),
            pltpu.VMEM((2 * 32,), jnp.float32),
            pltpu.VMEM((16,), jnp.int32),
            pltpu.VMEM((16,), jnp.int32),
        ],
    )(_scatter_body)


def _rne_bf16(x):
    bi = plsc.bitcast(x, jnp.int32)
    r = bi + 32767 + (lax.shift_right_logical(bi, 16) & 1)
    return plsc.bitcast(r & jnp.int32(-65536), jnp.float32)


def _scatter_body(cloud_hbm, mats_hbm, out_hbm, zbuf, xs, ys, zs, matv,
                  svec0, svec1):
    cid = lax.axis_index("c")
    sid = lax.axis_index("s")
    wid = cid * NS + sid
    iota = lax.broadcasted_iota(jnp.int32, (16,), 0)
    idxm1 = jnp.maximum(iota - 1, 0)
    maxv = jnp.full((16,), I32MAX, jnp.int32)
    svecs = (svec0, svec1)
    lanerev = 15 - iota

    pltpu.sync_copy(mats_hbm, matv)

    def init_body(i, _):
        for u in range(8):
            zbuf[pl.ds((i * 8 + u) * 16, 16)] = maxv
        return 0

    lax.fori_loop(0, SIZE // 128, init_body, 0)

    b = wid // NS
    boff = b * 32
    mm0 = matv[pl.ds(boff, 16)]
    mm1 = matv[pl.ds(boff + 16, 16)]
    m = [mm0[i] for i in range(16)] + [mm1[i] for i in range(12)]
    bhw = b * (H * W)
    base = wid * PTS_PER_W
    nbase = (wid % NS) * PTS_PER_W
    for ch in range(PTS_PER_W // CHUNK):
        n0 = nbase + ch * CHUNK
        pltpu.sync_copy(cloud_hbm.at[b, 0, pl.ds(n0, CHUNK)], xs)
        pltpu.sync_copy(cloud_hbm.at[b, 1, pl.ds(n0, CHUNK)], ys)
        pltpu.sync_copy(cloud_hbm.at[b, 2, pl.ds(n0, CHUNK)], zs)
        pftop = PN - 1 - base - ch * CHUNK

        def pt_body(i, _):
            for u in range(2):
                o = (i * 2 + u) * 16
                xb = _rne_bf16(xs[pl.ds(o, 16)])
                yb = _rne_bf16(ys[pl.ds(o, 16)])
                zb = _rne_bf16(zs[pl.ds(o, 16)])
                cam0 = xb * m[0] + yb * m[1] + zb * m[2] + m[3]
                cam1 = xb * m[4] + yb * m[5] + zb * m[6] + m[7]
                cam2 = xb * m[8] + yb * m[9] + zb * m[10] + m[11]
                c0 = _rne_bf16(cam0)
                c1 = _rne_bf16(cam1)
                c2 = _rne_bf16(cam2)
                p0 = c0 * m[16] + c1 * m[17] + c2 * m[18]
                p1 = c0 * m[20] + c1 * m[21] + c2 * m[22]
                p2 = c0 * m[24] + c1 * m[25] + c2 * m[26]
                uu = jnp.clip((p0 / p2).astype(jnp.int32), 0, W - 1)
                vv = jnp.clip((p1 / p2).astype(jnp.int32), 0, H - 1)
                pix = bhw + vv * W + uu
                di = cam2.astype(jnp.int32)
                sk = pix * 512 + jnp.clip(di, 0, 31) * 16 + lanerev
                fk = (jnp.clip(di, 0, 4094) * 524288
                      + (pftop - o) - iota)
                k_s, v_s = plsc.sort_key_val(sk, fk)
                svecs[u][...] = k_s
                prev = plsc.load_gather(svecs[u], [idxm1])
                spix = lax.shift_right_logical(k_s, 9)
                ppix = lax.shift_right_logical(prev, 9)
                first = (spix != ppix) | (iota == 0)
                cur = plsc.load_gather(zbuf, [spix])
                newv = jnp.minimum(cur, v_s)
                plsc.store_scatter(zbuf, [spix], newv, mask=first)
            return 0

        lax.fori_loop(0, CHUNK // 32, pt_body, 0)

    pltpu.sync_copy(zbuf, out_hbm.at[wid])


@functools.cache
def _build_winner_k():
    mesh = plsc.VectorSubcoreMesh(
        core_axis_name="c", subcore_axis_name="s",
        num_cores=NC, num_subcores=NS)
    return functools.partial(
        pl.kernel,
        out_type=jax.ShapeDtypeStruct((BQ * 3 * H * W,), jnp.float32),
        mesh=mesh,
        compiler_params=pltpu.CompilerParams(needs_layout_passes=False),
        scratch_types=[
            pltpu.VMEM((SLICE_O,), jnp.int32),
            pltpu.VMEM((SLICE_O,), jnp.int32),
            pltpu.VMEM((SLICE_O,), jnp.int32),
            pltpu.VMEM((SLICE_O,), jnp.int32),
            pltpu.VMEM((SLICE_O,), jnp.int32),
            pltpu.VMEM((SLICE_O,), jnp.int32),
            pltpu.VMEM((3 * SLICE_O,), jnp.float32),
            pltpu.VMEM((3 * SLICE_O,), jnp.float32),
            pltpu.SemaphoreType.DMA,
            pltpu.SemaphoreType.DMA,
            pltpu.SemaphoreType.DMA,
        ],
    )(_winner_body)


def _winner_body(p_hbm, rgb_hbm, img_hbm, k0, kA, kB, ix0, ix1, ix2, gat,
                 ost, semA, semB, semG):
    cid = lax.axis_index("c")
    sid = lax.axis_index("s")
    wid = cid * NS + sid
    pb = wid * SLICE_O
    b = wid // NS
    pltpu.sync_copy(p_hbm.at[0, pl.ds(pb, SLICE_O)], k0)
    bufs = (kA, kB)
    sems = (semA, semB)
    cps = [None, None]
    cps[1] = pltpu.async_copy(p_hbm.at[1, pl.ds(pb, SLICE_O)], bufs[1],
                              sems[1])
    for j in range(1, NW):
        if j + 1 < NW:
            nb = (j + 1) % 2
            cps[nb] = pltpu.async_copy(
                p_hbm.at[j + 1, pl.ds(pb, SLICE_O)], bufs[nb], sems[nb])
        cps[j % 2].wait()
        buf = bufs[j % 2]

        def merge_body(i, _):
            for u in range(4):
                o = (i * 4 + u) * 16
                k0[pl.ds(o, 16)] = jnp.minimum(
                    k0[pl.ds(o, 16)], buf[pl.ds(o, 16)])
            return 0

        lax.fori_loop(0, SLICE_O // 64, merge_body, 0)
    iota = lax.broadcasted_iota(jnp.int32, (16,), 0)
    ixs = (ix0, ix1, ix2)

    def idx_body(i, _):
        a = k0[pl.ds(i * 16, 16)]
        valid = a != I32MAX
        posf = a & POSMASK
        n = (PN - 1) - posf - b * NPTS
        spread = (pb + i * 16 + iota) & 1023
        base_i = b * (3 * NPTS) + n
        for c in range(3):
            ixs[c][pl.ds(i * 16, 16)] = jnp.where(
                valid, base_i + c * NPTS, spread)
        return 0

    lax.fori_loop(0, SLICE_O // 16, idx_body, 0)

    copies = [
        pltpu.async_copy(rgb_hbm.at[ixs[c]],
                         gat.at[pl.ds(c * SLICE_O, SLICE_O)], semG)
        for c in range(3)
    ]
    for cp in copies:
        cp.wait()

    def out_body(i, _):
        a = k0[pl.ds(i * 16, 16)]
        valid = a != I32MAX
        for c in range(3):
            g = gat[pl.ds(c * SLICE_O + i * 16, 16)]
            ost[pl.ds(c * SLICE_O + i * 16, 16)] = jnp.where(
                valid, g, -0.001)
        return 0

    lax.fori_loop(0, SLICE_O // 16, out_body, 0)

    pwb = pb - b * (H * W)
    for c in range(3):
        dst = b * (3 * H * W) + c * (H * W) + pwb
        pltpu.sync_copy(ost.at[pl.ds(c * SLICE_O, SLICE_O)],
                        img_hbm.at[pl.ds(dst, SLICE_O)])


def _median_body(img_ref, out_ref):
    img = img_ref[0]
    mask3 = jnp.broadcast_to(img[0:1] > 0, (3, H, W))

    def pad_reflect(x):
        x = jnp.concatenate(
            [x[:, 2:3], x[:, 1:2], x, x[:, H - 2:H - 1], x[:, H - 3:H - 2]],
            axis=1)
        x = jnp.concatenate(
            [x[:, :, 2:3], x[:, :, 1:2], x,
             x[:, :, W - 2:W - 1], x[:, :, W - 3:W - 2]],
            axis=2)
        return x

    def median25(x):
        p = pad_reflect(x).astype(jnp.bfloat16)
        v = [p[:, di:di + H, dj:dj + W] for di in range(5) for dj in range(5)]
        for kind, i, j in _MED_OPS:
            if kind == 'ce':
                v[i], v[j] = jnp.minimum(v[i], v[j]), jnp.maximum(v[i], v[j])
            elif kind == 'min':
                v[i] = jnp.minimum(v[i], v[j])
            else:
                v[j] = jnp.maximum(v[i], v[j])
        return v[12].astype(jnp.float32)

    def it_body(_, x):
        return jnp.where(mask3, x, median25(x))

    inp = lax.fori_loop(0, 10, it_body, img)

    anyv = ((inp[0] > 0) | (inp[1] > 0) | (inp[2] > 0)).astype(jnp.float32)
    zc = jnp.zeros((1, W), jnp.float32)
    zr = jnp.zeros((H + 2, 1), jnp.float32)
    p = jnp.concatenate([zc, anyv, zc], axis=0)
    p = jnp.concatenate([zr, p, zr], axis=1)
    neigh = p[0:H, 0:W]
    for di in range(3):
        for dj in range(3):
            if di == 0 and dj == 0:
                continue
            neigh = neigh + p[di:di + H, dj:dj + W]
    upd = (neigh >= 6.0).astype(jnp.float32)
    out_ref[0] = inp * upd[None]


def _median_call(img, interpret=False):
    return pl.pallas_call(
        _median_body,
        grid=(BQ,),
        in_specs=[pl.BlockSpec((1, 3, H, W), lambda b: (b, 0, 0, 0))],
        out_specs=pl.BlockSpec((1, 3, H, W), lambda b: (b, 0, 0, 0)),
        out_shape=jax.ShapeDtypeStruct((BQ, 3, H, W), jnp.float32),
        interpret=interpret,
    )(img)


def kernel(cloud, rgb_vec, K, T):
    Tinv = jnp.linalg.inv(T)
    Kpad = jnp.pad(K, ((0, 0), (0, 1), (0, 1)))
    mats = jnp.concatenate([Tinv, Kpad], axis=1)
    mats = mats.astype(jnp.bfloat16).astype(jnp.float32)
    planes = _build_scatter_k()(cloud, mats.reshape(BQ * 32))
    img_flat = _build_winner_k()(planes, rgb_vec.reshape(PN * 3))
    return _median_call(img_flat.reshape(BQ, 3, H, W))
